# Initial kernel scaffold; baseline (speedup 1.0000x reference)
import jax
import jax.numpy as jnp
from jax.experimental import pallas as pl


def kernel(prev_hidden, neigh_hidden, W):
    raise NotImplementedError("write your pallas kernel here")


# speedup vs baseline: 1.1794x; 1.1794x over previous
import jax
import jax.numpy as jnp
from jax.experimental import pallas as pl


def _body(prev_ref, neigh_ref, w_ref, out_ref):
    k = neigh_ref.shape[1]
    s = jnp.sum(neigh_ref[...], axis=1) + prev_ref[...]
    means = s * (1.0 / (k + 1))
    acc = jnp.dot(means, w_ref[...], preferred_element_type=jnp.float32)
    out_ref[...] = jnp.maximum(acc, 0.0)


def kernel(prev_hidden, neigh_hidden, W):
    n, d = prev_hidden.shape
    _, k, _ = neigh_hidden.shape
    f = W.shape[1]
    bn = 400
    grid = (n // bn,)
    return pl.pallas_call(
        _body,
        grid=grid,
        in_specs=[
            pl.BlockSpec((bn, d), lambda i: (i, 0)),
            pl.BlockSpec((bn, k, d), lambda i: (i, 0, 0)),
            pl.BlockSpec((d, f), lambda i: (0, 0)),
        ],
        out_specs=pl.BlockSpec((bn, f), lambda i: (i, 0)),
        out_shape=jax.ShapeDtypeStruct((n, f), jnp.float32),
    )(prev_hidden, neigh_hidden, W)


# speedup vs baseline: 1.2320x; 1.0445x over previous
import jax
import jax.numpy as jnp
from jax.experimental import pallas as pl


def _body(prev_ref, neigh_ref, w_ref, out_ref):
    k = neigh_ref.shape[1]
    s = jnp.sum(neigh_ref[...], axis=1) + prev_ref[...]
    means = s * (1.0 / (k + 1))
    acc = jnp.dot(means, w_ref[...], preferred_element_type=jnp.float32)
    out_ref[...] = jnp.maximum(acc, 0.0)


def kernel(prev_hidden, neigh_hidden, W):
    n, d = prev_hidden.shape
    _, k, _ = neigh_hidden.shape
    f = W.shape[1]
    bn = 1000
    grid = (n // bn,)
    return pl.pallas_call(
        _body,
        grid=grid,
        in_specs=[
            pl.BlockSpec((bn, d), lambda i: (i, 0)),
            pl.BlockSpec((bn, k, d), lambda i: (i, 0, 0)),
            pl.BlockSpec((d, f), lambda i: (0, 0)),
        ],
        out_specs=pl.BlockSpec((bn, f), lambda i: (i, 0)),
        out_shape=jax.ShapeDtypeStruct((n, f), jnp.float32),
    )(prev_hidden, neigh_hidden, W)
